# baseline (speedup 1.0000x reference)
import functools

import jax
import jax.numpy as jnp
from jax import lax
from jax.experimental import pallas as pl
from jax.experimental.pallas import tpu as pltpu
from jax.experimental.pallas import tpu_sc as plsc

B, V, C, S = 4096, 100, 64, 1000
BV = B * V
VS = V * S
NC, NS, L = 2, 16, 16
NW = NC * NS
PER = BV // NW
CHUNK = 128
NCHUNK = PER // CHUNK
NBUF = 10


@functools.partial(
    pl.kernel,
    out_type=jax.ShapeDtypeStruct((BV, C), jnp.bfloat16),
    mesh=plsc.VectorSubcoreMesh(core_axis_name="c", subcore_axis_name="s"),
    scratch_types=[
        pltpu.VMEM((PER,), jnp.int32),
        pltpu.VMEM((NBUF, CHUNK, C), jnp.bfloat16),
        pltpu.SemaphoreType.DMA((NBUF,)),
        pltpu.SemaphoreType.DMA,
    ],
    compiler_params=pltpu.CompilerParams(use_tc_tiling_on_sc=False),
)
def _sc_gather(x_hbm, tab_hbm, out_hbm, idx_v, rows_v, gsem, ssem):
    wid = lax.axis_index("s") * NC + lax.axis_index("c")
    base = wid * PER
    pltpu.sync_copy(x_hbm.at[pl.ds(base, PER)], idx_v)
    lane = lax.iota(jnp.int32, 16)

    def to_indices(r):
        rowbase = base + r * CHUNK
        for c in range(CHUNK // L):
            f = rowbase + c * L + lane
            xv = idx_v[pl.ds(r * CHUNK + c * L, L)]
            idx_v[pl.ds(r * CHUNK + c * L, L)] = (f % V) * S + xv

    def fire_gather(j, b):
        pltpu.async_copy(
            tab_hbm.at[idx_v.at[pl.ds(j * CHUNK, CHUNK)]], rows_v.at[b],
            gsem.at[b])

    def wait_gather(j, b):
        pltpu.make_async_copy(
            tab_hbm.at[idx_v.at[pl.ds(j * CHUNK, CHUNK)]], rows_v.at[b],
            gsem.at[b]).wait()

    def store(j, b):
        pltpu.async_copy(
            rows_v.at[b], out_hbm.at[pl.ds(base + j * CHUNK, CHUNK)], ssem
        ).wait()

    for b in range(NBUF):
        to_indices(b)
        fire_gather(b, b)

    def transform_rest(r, carry):
        to_indices(r)
        return carry

    lax.fori_loop(NBUF, NCHUNK, transform_rest, 0)

    def round_fn(gi, carry):
        g = gi * NBUF
        for b in range(NBUF):
            j = g + b
            wait_gather(j, b)
            store(j, b)
            fire_gather(j + NBUF, b)
        return carry

    lax.fori_loop(0, (NCHUNK - NBUF) // NBUF, round_fn, 0)

    for b in range(NBUF):
        j = NCHUNK - NBUF + b
        wait_gather(j, b)
        store(j, b)


def kernel(x, weight):
    tab = (
        jnp.transpose(weight.reshape(V, C, S), (0, 2, 1))
        .reshape(VS, C).astype(jnp.bfloat16))
    out = _sc_gather(x.reshape(BV), tab)
    return out.astype(jnp.float32).reshape(B, V, 1, C)


# speedup vs baseline: 1.6752x; 1.6752x over previous
import functools

import jax
import jax.numpy as jnp
from jax import lax
from jax.experimental import pallas as pl
from jax.experimental.pallas import tpu as pltpu
from jax.experimental.pallas import tpu_sc as plsc

B, V, C, S = 4096, 100, 64, 1000
BV = B * V
VS = V * S
NC, NS, L = 2, 16, 16
NW = NC * NS
PER = BV // NW
CHUNK = 64
NCHUNK = PER // CHUNK
RING = 20
LEAD = 10
assert (NCHUNK - 2 * LEAD) % RING == 0 and RING == 2 * LEAD


@functools.partial(
    pl.kernel,
    out_type=jax.ShapeDtypeStruct((BV, C), jnp.float32),
    mesh=plsc.VectorSubcoreMesh(core_axis_name="c", subcore_axis_name="s"),
    scratch_types=[
        pltpu.VMEM((PER,), jnp.int32),
        pltpu.VMEM((RING, CHUNK, C), jnp.float32),
        pltpu.SemaphoreType.DMA((RING,)),
        pltpu.SemaphoreType.DMA((LEAD,)),
    ],
    compiler_params=pltpu.CompilerParams(use_tc_tiling_on_sc=False),
)
def _sc_gather(x_hbm, tab_hbm, out_hbm, idx_v, rows_v, gsem, ssem):
    wid = lax.axis_index("s") * NC + lax.axis_index("c")
    base = wid * PER
    pltpu.sync_copy(x_hbm.at[pl.ds(base, PER)], idx_v)
    lane = lax.iota(jnp.int32, 16)

    def to_indices(r):
        rowbase = base + r * CHUNK
        for c in range(CHUNK // L):
            f = rowbase + c * L + lane
            xv = idx_v[pl.ds(r * CHUNK + c * L, L)]
            idx_v[pl.ds(r * CHUNK + c * L, L)] = (f % V) * S + xv

    def fire_gather(j, slot):
        pltpu.async_copy(
            tab_hbm.at[idx_v.at[pl.ds(j * CHUNK, CHUNK)]], rows_v.at[slot],
            gsem.at[slot])

    def wait_gather(j, slot):
        pltpu.make_async_copy(
            tab_hbm.at[idx_v.at[pl.ds(j * CHUNK, CHUNK)]], rows_v.at[slot],
            gsem.at[slot]).wait()

    def fire_store(j, slot, sslot):
        pltpu.async_copy(
            rows_v.at[slot], out_hbm.at[pl.ds(base + j * CHUNK, CHUNK)],
            ssem.at[sslot])

    def wait_store(j, slot, sslot):
        pltpu.make_async_copy(
            rows_v.at[slot], out_hbm.at[pl.ds(base + j * CHUNK, CHUNK)],
            ssem.at[sslot]).wait()

    for j in range(LEAD):
        to_indices(j)
        fire_gather(j, j)

    for j in range(LEAD):
        to_indices(j + LEAD)
        fire_gather(j + LEAD, j + LEAD)
        wait_gather(j, j)
        fire_store(j, j, j % LEAD)

    def round_fn(gi, carry):
        g = LEAD + gi * RING
        for b in range(RING):
            j = g + b
            slot = (LEAD + b) % RING
            to_indices(j + LEAD)
            wait_store(j - LEAD, (slot + LEAD) % RING, b % LEAD)
            fire_gather(j + LEAD, (slot + LEAD) % RING)
            wait_gather(j, slot)
            fire_store(j, slot, b % LEAD)
        return carry

    lax.fori_loop(0, (NCHUNK - 2 * LEAD) // RING, round_fn, 0)

    for j in range(NCHUNK - LEAD, NCHUNK):
        slot = j % RING
        wait_store(j - LEAD, (slot + LEAD) % RING, j % LEAD)
        wait_gather(j, slot)
        fire_store(j, slot, j % LEAD)

    for j in range(NCHUNK - LEAD, NCHUNK):
        wait_store(j, j % RING, j % LEAD)


def kernel(x, weight):
    tab = jnp.transpose(weight.reshape(V, C, S), (0, 2, 1)).reshape(VS, C)
    out = _sc_gather(x.reshape(BV), tab)
    return out.reshape(B, V, 1, C)
